# baseline (speedup 1.0000x reference)
import functools

import jax
import jax.numpy as jnp
from jax import lax
from jax.experimental import pallas as pl
from jax.experimental.pallas import tpu as pltpu
from jax.experimental.pallas import tpu_sc as plsc

EMB_DIM = 64
FIX_LEN = 32
H1 = 128
OUT = 2

_INFO = plsc.get_sparse_core_info()
_NC, _NS = _INFO.num_cores, _INFO.num_subcores
_NW = _NC * _NS

_CHUNK = 128
_NBUF = 4

_CONV_COLS = 8192


def _conv_body(xT_ref, tp_ref):
    tp_ref[:, 0:EMB_DIM] = xT_ref[...].T
    tp_ref[:, EMB_DIM : 2 * EMB_DIM] = jnp.zeros(
        (_CONV_COLS, EMB_DIM), jnp.float32
    )


def _tc_convert(emb_table):
    V = emb_table.shape[0]
    xT = emb_table.T
    n_blk = (V + _CONV_COLS - 1) // _CONV_COLS
    return pl.pallas_call(
        _conv_body,
        grid=(n_blk,),
        in_specs=[pl.BlockSpec((EMB_DIM, _CONV_COLS), lambda i: (0, i))],
        out_specs=pl.BlockSpec((_CONV_COLS, 2 * EMB_DIM), lambda i: (i, 0)),
        out_shape=jax.ShapeDtypeStruct((n_blk * _CONV_COLS, 2 * EMB_DIM), jnp.float32),
    )(xT)


def _gather_body(
    rows_per_w,
    texts_hbm,
    tp_hbm,
    out_hbm,
    idx_v,
    r0,
    r1,
    r2,
    r3,
    si0,
    si1,
    si2,
    si3,
    so0,
    so1,
    so2,
    so3,
):
    wid = lax.axis_index("s") * _NC + lax.axis_index("c")
    base = wid * rows_per_w
    pltpu.sync_copy(texts_hbm.at[pl.ds(base, rows_per_w)], idx_v)
    n_chunks = rows_per_w // _CHUNK
    bufs = (r0, r1, r2, r3)
    isems = (si0, si1, si2, si3)
    osems = (so0, so1, so2, so3)
    last = n_chunks - 1

    def fire_in(g, b):
        pltpu.async_copy(
            tp_hbm.at[idx_v.at[pl.ds(g * _CHUNK, _CHUNK)]], bufs[b], isems[b]
        )

    def wait_in(b):
        pltpu.make_async_copy(
            tp_hbm.at[idx_v.at[pl.ds(0, _CHUNK)]], bufs[b], isems[b]
        ).wait()

    def fire_out(g, b):
        pltpu.async_copy(
            bufs[b], out_hbm.at[pl.ds(base + g * _CHUNK, _CHUNK)], osems[b]
        )

    def wait_out(b):
        pltpu.make_async_copy(
            bufs[b], out_hbm.at[pl.ds(base, _CHUNK)], osems[b]
        ).wait()

    for b in range(_NBUF):
        fire_in(b, b)

    def outer(j, carry):
        g0 = j * _NBUF
        for b in range(_NBUF):
            wait_in(b)
            fire_out(g0 + b, b)
        for b in range(_NBUF):
            wait_out(b)
            fire_in(jnp.minimum(g0 + b + _NBUF, last), b)
        return carry

    lax.fori_loop(0, n_chunks // _NBUF, outer, 0, unroll=False)
    for b in range(_NBUF):
        wait_in(b)


def _sc_gather(idx, tp):
    n_rows = idx.shape[0]
    rows_per_w = n_rows // _NW
    mesh = plsc.VectorSubcoreMesh(core_axis_name="c", subcore_axis_name="s")
    buf = pltpu.VMEM((_CHUNK, 2 * EMB_DIM), jnp.float32)
    k = functools.partial(
        pl.kernel,
        out_type=jax.ShapeDtypeStruct((n_rows, 2 * EMB_DIM), jnp.float32),
        mesh=mesh,
        scratch_types=[pltpu.VMEM((rows_per_w,), jnp.int32)]
        + [buf] * _NBUF
        + [pltpu.SemaphoreType.DMA] * (2 * _NBUF),
    )(functools.partial(_gather_body, rows_per_w))
    return k(idx, tp)


def _mlp_body(x_ref, w1_ref, b1_ref, w2_ref, b2_ref, o_ref):
    bm = x_ref.shape[0]
    x = x_ref[...].reshape(bm, FIX_LEN * 2 * EMB_DIM)
    h = jnp.dot(x, w1_ref[...], preferred_element_type=jnp.float32)
    h = h + b1_ref[...]
    h = jnp.where(h >= 0, h, 0.01 * h)
    o_ref[...] = jnp.dot(h, w2_ref[...], preferred_element_type=jnp.float32) + b2_ref[...]


def _tc_mlp(x3, W1z, b1, W2, b2):
    B = x3.shape[0]
    K = FIX_LEN * 2 * EMB_DIM
    BM = 1024
    return pl.pallas_call(
        _mlp_body,
        grid=(B // BM,),
        in_specs=[
            pl.BlockSpec((BM, FIX_LEN, 2 * EMB_DIM), lambda i: (i, 0, 0)),
            pl.BlockSpec((K, H1), lambda i: (0, 0)),
            pl.BlockSpec((1, H1), lambda i: (0, 0)),
            pl.BlockSpec((H1, OUT), lambda i: (0, 0)),
            pl.BlockSpec((1, OUT), lambda i: (0, 0)),
        ],
        out_specs=pl.BlockSpec((BM, OUT), lambda i: (i, 0)),
        out_shape=jax.ShapeDtypeStruct((B, OUT), jnp.float32),
    )(x3, W1z, b1.reshape(1, H1), W2, b2.reshape(1, OUT))


_N_CHUNKS = 4


def kernel(texts, emb_table, W1, b1, W2, b2):
    B, L = texts.shape
    texts_flat = texts.reshape(-1).astype(jnp.int32)
    tp = _tc_convert(emb_table)
    w1r = W1.reshape(FIX_LEN, EMB_DIM, H1)
    W1z = jnp.pad(w1r, ((0, 0), (0, EMB_DIM), (0, 0))).reshape(
        FIX_LEN * 2 * EMB_DIM, H1
    )
    bc = B // _N_CHUNKS
    outs = []
    for c in range(_N_CHUNKS):
        idx_c = lax.dynamic_slice_in_dim(texts_flat, c * bc * L, bc * L)
        fat = _sc_gather(idx_c, tp)
        x3 = fat.reshape(bc, L, 2 * EMB_DIM)
        outs.append(_tc_mlp(x3, W1z, b1, W2, b2))
    return jnp.concatenate(outs, axis=0)


# speedup vs baseline: 1.0246x; 1.0246x over previous
import functools

import jax
import jax.numpy as jnp
from jax import lax
from jax.experimental import pallas as pl
from jax.experimental.pallas import tpu as pltpu
from jax.experimental.pallas import tpu_sc as plsc

EMB_DIM = 64
FIX_LEN = 32
H1 = 128
OUT = 2

_INFO = plsc.get_sparse_core_info()
_NC, _NS = _INFO.num_cores, _INFO.num_subcores
_NW = _NC * _NS

_CHUNK = 128
_NBUF = 4

_CONV_COLS = 8192


def _conv_body(xT_ref, tp_ref):
    tp_ref[:, 0:EMB_DIM] = xT_ref[...].T
    tp_ref[:, EMB_DIM : 2 * EMB_DIM] = jnp.zeros(
        (_CONV_COLS, EMB_DIM), jnp.float32
    )


def _tc_convert(emb_table):
    V = emb_table.shape[0]
    xT = emb_table.T
    n_blk = (V + _CONV_COLS - 1) // _CONV_COLS
    return pl.pallas_call(
        _conv_body,
        grid=(n_blk,),
        in_specs=[pl.BlockSpec((EMB_DIM, _CONV_COLS), lambda i: (0, i))],
        out_specs=pl.BlockSpec((_CONV_COLS, 2 * EMB_DIM), lambda i: (i, 0)),
        out_shape=jax.ShapeDtypeStruct((n_blk * _CONV_COLS, 2 * EMB_DIM), jnp.float32),
    )(xT)


def _gather_body(
    rows_per_w,
    texts_hbm,
    tp_hbm,
    out_hbm,
    idx_v,
    r0,
    r1,
    r2,
    r3,
    si0,
    si1,
    si2,
    si3,
    so0,
    so1,
    so2,
    so3,
):
    wid = lax.axis_index("s") * _NC + lax.axis_index("c")
    base = wid * rows_per_w
    pltpu.sync_copy(texts_hbm.at[pl.ds(base, rows_per_w)], idx_v)
    n_chunks = rows_per_w // _CHUNK
    bufs = (r0, r1, r2, r3)
    isems = (si0, si1, si2, si3)
    osems = (so0, so1, so2, so3)
    last = n_chunks - 1

    def fire_in(g, b):
        pltpu.async_copy(
            tp_hbm.at[idx_v.at[pl.ds(g * _CHUNK, _CHUNK)]], bufs[b], isems[b]
        )

    def wait_in(b):
        pltpu.make_async_copy(
            tp_hbm.at[idx_v.at[pl.ds(0, _CHUNK)]], bufs[b], isems[b]
        ).wait()

    def fire_out(g, b):
        pltpu.async_copy(
            bufs[b], out_hbm.at[pl.ds(base + g * _CHUNK, _CHUNK)], osems[b]
        )

    def wait_out(b):
        pltpu.make_async_copy(
            bufs[b], out_hbm.at[pl.ds(base, _CHUNK)], osems[b]
        ).wait()

    for b in range(_NBUF):
        fire_in(b, b)

    def outer(j, carry):
        g0 = j * _NBUF
        for b in range(_NBUF):
            wait_in(b)
            fire_out(g0 + b, b)
        for b in range(_NBUF):
            wait_out(b)
            fire_in(jnp.minimum(g0 + b + _NBUF, last), b)
        return carry

    lax.fori_loop(0, n_chunks // _NBUF, outer, 0, unroll=False)
    for b in range(_NBUF):
        wait_in(b)


def _sc_gather(idx, tp):
    n_rows = idx.shape[0]
    rows_per_w = n_rows // _NW
    mesh = plsc.VectorSubcoreMesh(core_axis_name="c", subcore_axis_name="s")
    buf = pltpu.VMEM((_CHUNK, 2 * EMB_DIM), jnp.float32)
    k = functools.partial(
        pl.kernel,
        out_type=jax.ShapeDtypeStruct((n_rows, 2 * EMB_DIM), jnp.float32),
        mesh=mesh,
        scratch_types=[pltpu.VMEM((rows_per_w,), jnp.int32)]
        + [buf] * _NBUF
        + [pltpu.SemaphoreType.DMA] * (2 * _NBUF),
    )(functools.partial(_gather_body, rows_per_w))
    return k(idx, tp)


def _mlp_body(x_ref, w1_ref, b1_ref, w2_ref, b2_ref, o_ref):
    bm = x_ref.shape[0]
    x = x_ref[...].reshape(bm, FIX_LEN * 2 * EMB_DIM)
    h = jnp.dot(x, w1_ref[...], preferred_element_type=jnp.float32)
    h = h + b1_ref[...]
    h = jnp.where(h >= 0, h, 0.01 * h)
    o_ref[...] = jnp.dot(h, w2_ref[...], preferred_element_type=jnp.float32) + b2_ref[...]


def _tc_mlp(x3, W1z, b1, W2, b2):
    B = x3.shape[0]
    K = FIX_LEN * 2 * EMB_DIM
    BM = 1024
    return pl.pallas_call(
        _mlp_body,
        grid=(B // BM,),
        in_specs=[
            pl.BlockSpec((BM, FIX_LEN, 2 * EMB_DIM), lambda i: (i, 0, 0)),
            pl.BlockSpec((K, H1), lambda i: (0, 0)),
            pl.BlockSpec((1, H1), lambda i: (0, 0)),
            pl.BlockSpec((H1, OUT), lambda i: (0, 0)),
            pl.BlockSpec((1, OUT), lambda i: (0, 0)),
        ],
        out_specs=pl.BlockSpec((BM, OUT), lambda i: (i, 0)),
        out_shape=jax.ShapeDtypeStruct((B, OUT), jnp.float32),
    )(x3, W1z, b1.reshape(1, H1), W2, b2.reshape(1, OUT))


_N_CHUNKS = 2


def kernel(texts, emb_table, W1, b1, W2, b2):
    B, L = texts.shape
    texts_flat = texts.reshape(-1).astype(jnp.int32)
    tp = _tc_convert(emb_table)
    w1r = W1.reshape(FIX_LEN, EMB_DIM, H1)
    W1z = jnp.pad(w1r, ((0, 0), (0, EMB_DIM), (0, 0))).reshape(
        FIX_LEN * 2 * EMB_DIM, H1
    )
    bc = B // _N_CHUNKS
    outs = []
    for c in range(_N_CHUNKS):
        idx_c = lax.dynamic_slice_in_dim(texts_flat, c * bc * L, bc * L)
        fat = _sc_gather(idx_c, tp)
        x3 = fat.reshape(bc, L, 2 * EMB_DIM)
        outs.append(_tc_mlp(x3, W1z, b1, W2, b2))
    return jnp.concatenate(outs, axis=0)


# speedup vs baseline: 1.0504x; 1.0252x over previous
import functools

import jax
import jax.numpy as jnp
from jax import lax
from jax.experimental import pallas as pl
from jax.experimental.pallas import tpu as pltpu
from jax.experimental.pallas import tpu_sc as plsc

EMB_DIM = 64
FIX_LEN = 32
H1 = 128
OUT = 2

_INFO = plsc.get_sparse_core_info()
_NC, _NS = _INFO.num_cores, _INFO.num_subcores
_NW = _NC * _NS

_CHUNK = 128
_NBUF = 4

_CONV_COLS = 8192


def _conv_body(xT_ref, tp_ref):
    tp_ref[:, 0:EMB_DIM] = xT_ref[...].T
    tp_ref[:, EMB_DIM : 2 * EMB_DIM] = jnp.zeros(
        (_CONV_COLS, EMB_DIM), jnp.float32
    )


def _tc_convert(emb_table):
    V = emb_table.shape[0]
    xT = emb_table.T
    n_blk = (V + _CONV_COLS - 1) // _CONV_COLS
    return pl.pallas_call(
        _conv_body,
        grid=(n_blk,),
        in_specs=[pl.BlockSpec((EMB_DIM, _CONV_COLS), lambda i: (0, i))],
        out_specs=pl.BlockSpec((_CONV_COLS, 2 * EMB_DIM), lambda i: (i, 0)),
        out_shape=jax.ShapeDtypeStruct((n_blk * _CONV_COLS, 2 * EMB_DIM), jnp.float32),
    )(xT)


def _gather_body(
    rows_per_w,
    texts_hbm,
    tp_hbm,
    out_hbm,
    idx_v,
    r0,
    r1,
    r2,
    r3,
    si0,
    si1,
    si2,
    si3,
    so0,
    so1,
    so2,
    so3,
):
    wid = lax.axis_index("s") * _NC + lax.axis_index("c")
    base = wid * rows_per_w
    pltpu.sync_copy(texts_hbm.at[pl.ds(base, rows_per_w)], idx_v)
    n_chunks = rows_per_w // _CHUNK
    bufs = (r0, r1, r2, r3)
    isems = (si0, si1, si2, si3)
    osems = (so0, so1, so2, so3)
    last = n_chunks - 1

    def fire_in(g, b):
        pltpu.async_copy(
            tp_hbm.at[idx_v.at[pl.ds(g * _CHUNK, _CHUNK)]], bufs[b], isems[b]
        )

    def wait_in(b):
        pltpu.make_async_copy(
            tp_hbm.at[idx_v.at[pl.ds(0, _CHUNK)]], bufs[b], isems[b]
        ).wait()

    def fire_out(g, b):
        pltpu.async_copy(
            bufs[b], out_hbm.at[pl.ds(base + g * _CHUNK, _CHUNK)], osems[b]
        )

    def wait_out(b):
        pltpu.make_async_copy(
            bufs[b], out_hbm.at[pl.ds(base, _CHUNK)], osems[b]
        ).wait()

    for b in range(_NBUF):
        fire_in(b, b)

    def outer(j, carry):
        g0 = j * _NBUF
        for b in range(_NBUF):
            wait_in(b)
            fire_out(g0 + b, b)
        for b in range(_NBUF):
            wait_out(b)
            fire_in(jnp.minimum(g0 + b + _NBUF, last), b)
        return carry

    lax.fori_loop(0, n_chunks // _NBUF, outer, 0, unroll=False)
    for b in range(_NBUF):
        wait_in(b)


def _sc_gather(idx, tp):
    n_rows = idx.shape[0]
    rows_per_w = n_rows // _NW
    mesh = plsc.VectorSubcoreMesh(core_axis_name="c", subcore_axis_name="s")
    buf = pltpu.VMEM((_CHUNK, 2 * EMB_DIM), jnp.float32)
    k = functools.partial(
        pl.kernel,
        out_type=jax.ShapeDtypeStruct((n_rows, 2 * EMB_DIM), jnp.float32),
        mesh=mesh,
        scratch_types=[pltpu.VMEM((rows_per_w,), jnp.int32)]
        + [buf] * _NBUF
        + [pltpu.SemaphoreType.DMA] * (2 * _NBUF),
    )(functools.partial(_gather_body, rows_per_w))
    return k(idx, tp)


def _mlp_body(x_ref, w1_ref, b1_ref, w2_ref, b2_ref, o_ref):
    bm = x_ref.shape[0]
    x = x_ref[...].reshape(bm, FIX_LEN * 2 * EMB_DIM)
    h = jnp.dot(x, w1_ref[...], preferred_element_type=jnp.float32)
    h = h + b1_ref[...]
    h = jnp.where(h >= 0, h, 0.01 * h)
    o_ref[...] = jnp.dot(h, w2_ref[...], preferred_element_type=jnp.float32) + b2_ref[...]


def _tc_mlp(x3, W1z, b1, W2, b2):
    B = x3.shape[0]
    K = FIX_LEN * 2 * EMB_DIM
    BM = 1024
    return pl.pallas_call(
        _mlp_body,
        grid=(B // BM,),
        in_specs=[
            pl.BlockSpec((BM, FIX_LEN, 2 * EMB_DIM), lambda i: (i, 0, 0)),
            pl.BlockSpec((K, H1), lambda i: (0, 0)),
            pl.BlockSpec((1, H1), lambda i: (0, 0)),
            pl.BlockSpec((H1, OUT), lambda i: (0, 0)),
            pl.BlockSpec((1, OUT), lambda i: (0, 0)),
        ],
        out_specs=pl.BlockSpec((BM, OUT), lambda i: (i, 0)),
        out_shape=jax.ShapeDtypeStruct((B, OUT), jnp.float32),
    )(x3, W1z, b1.reshape(1, H1), W2, b2.reshape(1, OUT))


_N_CHUNKS = 1


def kernel(texts, emb_table, W1, b1, W2, b2):
    B, L = texts.shape
    texts_flat = texts.reshape(-1).astype(jnp.int32)
    tp = _tc_convert(emb_table)
    w1r = W1.reshape(FIX_LEN, EMB_DIM, H1)
    W1z = jnp.pad(w1r, ((0, 0), (0, EMB_DIM), (0, 0))).reshape(
        FIX_LEN * 2 * EMB_DIM, H1
    )
    bc = B // _N_CHUNKS
    outs = []
    for c in range(_N_CHUNKS):
        idx_c = lax.dynamic_slice_in_dim(texts_flat, c * bc * L, bc * L)
        fat = _sc_gather(idx_c, tp)
        x3 = fat.reshape(bc, L, 2 * EMB_DIM)
        outs.append(_tc_mlp(x3, W1z, b1, W2, b2))
    return jnp.concatenate(outs, axis=0)
